# Initial kernel scaffold; baseline (speedup 1.0000x reference)
import jax
import jax.numpy as jnp
from jax.experimental import pallas as pl


def kernel(pred_disp):
    raise NotImplementedError("write your pallas kernel here")


# speedup vs baseline: 2.1125x; 2.1125x over previous
import jax
import jax.numpy as jnp
from jax import lax
from jax.experimental import pallas as pl
from jax.experimental.pallas import tpu as pltpu
from jax.experimental.pallas import tpu_sc as plsc

N, C, H, W = 8, 2, 256, 256
HW = H * W
NC, NS, L = 2, 16, 16
B_PER_SC = N // NC
PIX_SC = B_PER_SC * HW
PIX_TILE = PIX_SC // NS
CH = 4096
NCHUNK = PIX_TILE // CH
NUM_IT = 4


def _sc_body(pred_hbm, disp_out, cnt_out, lidx_out,
             tabx, taby, counts_sh,
             mx, my, gx, gy, idx0, idx1, idx2, idx3, onesb, sem0, sem1):
    c = lax.axis_index("c")
    s = lax.axis_index("s")
    b_local = s // 4
    b = c * B_PER_SC + b_local
    poff = (s % 4) * PIX_TILE
    ybase = (s % 4) * (PIX_TILE // W)
    lane = lax.broadcasted_iota(jnp.int32, (L,), 0)
    tbase = s * PIX_TILE
    idxs = (idx0, idx1, idx2, idx3)

    pltpu.sync_copy(pred_hbm.at[b, 0, pl.ds(poff, PIX_TILE)], mx)
    pltpu.sync_copy(pred_hbm.at[b, 1, pl.ds(poff, PIX_TILE)], my)
    pltpu.sync_copy(mx, tabx.at[pl.ds(tbase, PIX_TILE)])
    pltpu.sync_copy(my, taby.at[pl.ds(tbase, PIX_TILE)])

    def _fill(i, _):
        idx0[pl.ds(i * L, L)] = jnp.zeros((L,), jnp.int32)
        onesb[pl.ds(i * L, L)] = jnp.ones((L,), jnp.int32)
        return _
    lax.fori_loop(0, CH // L, _fill, None)
    for k in range(NCHUNK):
        pltpu.sync_copy(idx0, counts_sh.at[pl.ds(tbase + k * CH, CH)])
    plsc.subcore_barrier()

    for t in range(NUM_IT):
        last = t == NUM_IT - 1

        for k in range(NCHUNK):
            idxr = idxs[k]

            def _index(i, _):
                off = k * CH + i * L
                p = off + lane
                dx = mx[pl.ds(off, L)]
                dy = my[pl.ds(off, L)]
                xi = p & (W - 1)
                yi = ybase + (p >> 8)
                cxv = (xi.astype(jnp.float32) + dx).astype(jnp.int32)
                cyv = (yi.astype(jnp.float32) + dy).astype(jnp.int32)
                cxv = jnp.minimum(jnp.maximum(cxv, 0), W - 1)
                cyv = jnp.minimum(jnp.maximum(cyv, 0), H - 1)
                lidx = b_local * HW + cyv * W + cxv
                idxr[pl.ds(i * L, L)] = lidx
                return _
            lax.fori_loop(0, CH // L, _index, None, unroll=2)

        for k in range(NCHUNK):
            cpx = pltpu.async_copy(tabx.at[idxs[k]], gx, sem0)
            cpy = pltpu.async_copy(taby.at[idxs[k]], gy, sem1)
            cpx.wait()
            cpy.wait()

            def _accum(i, _):
                off = k * CH + i * L
                mx[pl.ds(off, L)] = mx[pl.ds(off, L)] + gx[pl.ds(i * L, L)]
                my[pl.ds(off, L)] = my[pl.ds(off, L)] + gy[pl.ds(i * L, L)]
                return _
            lax.fori_loop(0, CH // L, _accum, None, unroll=2)

        if not last:
            plsc.subcore_barrier()
            pltpu.sync_copy(mx, tabx.at[pl.ds(tbase, PIX_TILE)])
            pltpu.sync_copy(my, taby.at[pl.ds(tbase, PIX_TILE)])
            plsc.subcore_barrier()
        else:
            for k in range(NCHUNK):
                pltpu.sync_copy(onesb, counts_sh.at[idxs[k]], add=True)
            pltpu.sync_copy(mx, disp_out.at[b, 0, pl.ds(poff, PIX_TILE)])
            pltpu.sync_copy(my, disp_out.at[b, 1, pl.ds(poff, PIX_TILE)])
            for k in range(NCHUNK):
                pltpu.sync_copy(
                    idxs[k], lidx_out.at[b, pl.ds(poff + k * CH, CH)])
            plsc.subcore_barrier()
            pltpu.sync_copy(counts_sh.at[pl.ds(tbase, PIX_TILE)],
                            cnt_out.at[b, pl.ds(poff, PIX_TILE)])


@jax.jit
def _sc_iterate(pred):
    call = pl.kernel(
        _sc_body,
        mesh=plsc.VectorSubcoreMesh(core_axis_name="c", subcore_axis_name="s"),
        out_type=(
            jax.ShapeDtypeStruct((N, C, HW), jnp.float32),
            jax.ShapeDtypeStruct((N, HW), jnp.int32),
            jax.ShapeDtypeStruct((N, HW), jnp.int32),
        ),
        scratch_types=[
            pltpu.VMEM_SHARED((PIX_SC,), jnp.float32),
            pltpu.VMEM_SHARED((PIX_SC,), jnp.float32),
            pltpu.VMEM_SHARED((PIX_SC,), jnp.int32),
            pltpu.VMEM((PIX_TILE,), jnp.float32),
            pltpu.VMEM((PIX_TILE,), jnp.float32),
            pltpu.VMEM((CH,), jnp.float32),
            pltpu.VMEM((CH,), jnp.float32),
            pltpu.VMEM((CH,), jnp.int32),
            pltpu.VMEM((CH,), jnp.int32),
            pltpu.VMEM((CH,), jnp.int32),
            pltpu.VMEM((CH,), jnp.int32),
            pltpu.VMEM((CH,), jnp.int32),
            pltpu.SemaphoreType.DMA,
            pltpu.SemaphoreType.DMA,
        ],
    )
    return call(pred)


def kernel(pred_disp):
    pred = pred_disp.reshape(N, C, HW)
    disp, cnt, lidx = _sc_iterate(pred)
    disp_out = disp.reshape(N, C, H, W)
    num_touch = cnt.reshape(N, H, W)
    cx = (lidx & (W - 1)).reshape(N, H, W)
    cy = ((lidx >> 8) & (H - 1)).reshape(N, H, W)
    b_idx = jnp.broadcast_to(
        jnp.arange(N, dtype=jnp.int32)[:, None, None], (N, H, W))
    result_cent = jnp.stack([b_idx, cx, cy], axis=1)
    return disp_out, num_touch, result_cent


# speedup vs baseline: 4.5042x; 2.1322x over previous
import jax
import jax.numpy as jnp
from jax import lax
from jax.experimental import pallas as pl
from jax.experimental.pallas import tpu as pltpu
from jax.experimental.pallas import tpu_sc as plsc

N, C, H, W = 8, 2, 256, 256
HW = H * W
NC, NS, L = 2, 16, 16
B_PER_SC = N // NC
PIX_SC = B_PER_SC * HW
PIX_TILE = PIX_SC // NS
CH = 4096
NCHUNK = PIX_TILE // CH
NUM_IT = 4


def _sc_body(pred_hbm, disp_out, cnt_out, lidx_out,
             tabx, taby, counts_sh,
             mx, my, gx0, gy0, gx1, gy1,
             idx0, idx1, idx2, idx3, onesb, sem0, sem1):
    c = lax.axis_index("c")
    s = lax.axis_index("s")
    b_local = s // 4
    b = c * B_PER_SC + b_local
    poff = (s % 4) * PIX_TILE
    ybase = (s % 4) * (PIX_TILE // W)
    lane = lax.broadcasted_iota(jnp.int32, (L,), 0)
    tbase = s * PIX_TILE
    idxs = (idx0, idx1, idx2, idx3)
    gbufs = ((gx0, gy0), (gx1, gy1))

    pltpu.sync_copy(pred_hbm.at[b, 0, pl.ds(poff, PIX_TILE)], mx)
    pltpu.sync_copy(pred_hbm.at[b, 1, pl.ds(poff, PIX_TILE)], my)
    pltpu.sync_copy(mx, tabx.at[pl.ds(tbase, PIX_TILE)])
    pltpu.sync_copy(my, taby.at[pl.ds(tbase, PIX_TILE)])

    def _fill(i, _):
        idx0[pl.ds(i * L, L)] = jnp.zeros((L,), jnp.int32)
        onesb[pl.ds(i * L, L)] = jnp.ones((L,), jnp.int32)
        return _
    lax.fori_loop(0, CH // L, _fill, None)
    for k in range(NCHUNK):
        pltpu.sync_copy(idx0, counts_sh.at[pl.ds(tbase + k * CH, CH)])
    plsc.subcore_barrier()

    for t in range(NUM_IT):
        last = t == NUM_IT - 1

        def _index(k):
            idxr = idxs[k]

            def body(i, _):
                off = k * CH + i * L
                p = off + lane
                dx = mx[pl.ds(off, L)]
                dy = my[pl.ds(off, L)]
                xi = p & (W - 1)
                yi = ybase + (p >> 8)
                cxv = (xi.astype(jnp.float32) + dx).astype(jnp.int32)
                cyv = (yi.astype(jnp.float32) + dy).astype(jnp.int32)
                cxv = jnp.minimum(jnp.maximum(cxv, 0), W - 1)
                cyv = jnp.minimum(jnp.maximum(cyv, 0), H - 1)
                lidx = b_local * HW + cyv * W + cxv
                idxr[pl.ds(i * L, L)] = lidx
                return _
            lax.fori_loop(0, CH // L, body, None, unroll=4)
            if last:
                pltpu.sync_copy(onesb, counts_sh.at[idxs[k]], add=True)
                pltpu.sync_copy(
                    idxs[k], lidx_out.at[b, pl.ds(poff + k * CH, CH)])

        def _fire(k):
            gxk, gyk = gbufs[k % 2]
            cpx = pltpu.async_copy(tabx.at[idxs[k]], gxk, sem0)
            cpy = pltpu.async_copy(taby.at[idxs[k]], gyk, sem1)
            return cpx, cpy

        def _accum(k, cps):
            gxk, gyk = gbufs[k % 2]
            cps[0].wait()
            cps[1].wait()

            def body(i, _):
                off = k * CH + i * L
                mx[pl.ds(off, L)] = mx[pl.ds(off, L)] + gxk[pl.ds(i * L, L)]
                my[pl.ds(off, L)] = my[pl.ds(off, L)] + gyk[pl.ds(i * L, L)]
                return _
            lax.fori_loop(0, CH // L, body, None, unroll=4)

        _index(0)
        cp0 = _fire(0)
        _index(1)
        cp1 = _fire(1)
        _accum(0, cp0)
        _index(2)
        cp2 = _fire(2)
        _accum(1, cp1)
        _index(3)
        cp3 = _fire(3)
        _accum(2, cp2)
        _accum(3, cp3)

        if not last:
            plsc.subcore_barrier()
            pltpu.sync_copy(mx, tabx.at[pl.ds(tbase, PIX_TILE)])
            pltpu.sync_copy(my, taby.at[pl.ds(tbase, PIX_TILE)])
            plsc.subcore_barrier()
        else:
            pltpu.sync_copy(mx, disp_out.at[b, 0, pl.ds(poff, PIX_TILE)])
            pltpu.sync_copy(my, disp_out.at[b, 1, pl.ds(poff, PIX_TILE)])
            plsc.subcore_barrier()
            pltpu.sync_copy(counts_sh.at[pl.ds(tbase, PIX_TILE)],
                            cnt_out.at[b, pl.ds(poff, PIX_TILE)])


@jax.jit
def _sc_iterate(pred):
    call = pl.kernel(
        _sc_body,
        mesh=plsc.VectorSubcoreMesh(core_axis_name="c", subcore_axis_name="s"),
        out_type=(
            jax.ShapeDtypeStruct((N, C, HW), jnp.float32),
            jax.ShapeDtypeStruct((N, HW), jnp.int32),
            jax.ShapeDtypeStruct((N, HW), jnp.int32),
        ),
        scratch_types=[
            pltpu.VMEM_SHARED((PIX_SC,), jnp.float32),
            pltpu.VMEM_SHARED((PIX_SC,), jnp.float32),
            pltpu.VMEM_SHARED((PIX_SC,), jnp.int32),
            pltpu.VMEM((PIX_TILE,), jnp.float32),
            pltpu.VMEM((PIX_TILE,), jnp.float32),
            pltpu.VMEM((CH,), jnp.float32),
            pltpu.VMEM((CH,), jnp.float32),
            pltpu.VMEM((CH,), jnp.float32),
            pltpu.VMEM((CH,), jnp.float32),
            pltpu.VMEM((CH,), jnp.int32),
            pltpu.VMEM((CH,), jnp.int32),
            pltpu.VMEM((CH,), jnp.int32),
            pltpu.VMEM((CH,), jnp.int32),
            pltpu.VMEM((CH,), jnp.int32),
            pltpu.SemaphoreType.DMA,
            pltpu.SemaphoreType.DMA,
        ],
    )
    return call(pred)


def kernel(pred_disp):
    pred = pred_disp.reshape(N, C, HW)
    disp, cnt, lidx = _sc_iterate(pred)
    disp_out = disp.reshape(N, C, H, W)
    num_touch = cnt.reshape(N, H, W)
    cx = (lidx & (W - 1)).reshape(N, H, W)
    cy = ((lidx >> 8) & (H - 1)).reshape(N, H, W)
    b_idx = jnp.broadcast_to(
        jnp.arange(N, dtype=jnp.int32)[:, None, None], (N, H, W))
    result_cent = jnp.stack([b_idx, cx, cy], axis=1)
    return disp_out, num_touch, result_cent
